# Initial kernel scaffold; baseline (speedup 1.0000x reference)
import functools

import jax
import jax.numpy as jnp
from jax import lax
from jax.experimental import pallas as pl
from jax.experimental.pallas import tpu as pltpu
from jax.experimental.pallas import tpu_sc as plsc

B, L, HID = 4096, 200, 64
N = B * L
NC, NS = 2, 16
NW = NC * NS
TOK_PER_W = N // NW
C = 128
NCHUNK = TOK_PER_W // C
EPS = 1e-12
K = HID // 16


def _rsqrt_nr(v):
    i = plsc.bitcast(v, jnp.int32)
    i = jnp.int32(0x5F3759DF) - lax.shift_right_logical(i, jnp.int32(1))
    y = plsc.bitcast(i, jnp.float32)
    for _ in range(3):
        y = y * (1.5 - 0.5 * v * y * y)
    return y


def _body(tok_ids, pos_ids, tok_table, pos_table, ln_w, ln_b, out,
          idx_v, pidx_v, rows_v, prow_v, out_v, w_v, b_v, sem):
    c = lax.axis_index("c")
    s = lax.axis_index("s")
    wid = s * NC + c
    base0 = wid * TOK_PER_W

    pltpu.sync_copy(ln_w, w_v)
    pltpu.sync_copy(ln_b, b_v)
    wvec = [w_v[pl.ds(k * 16, 16)] for k in range(K)]
    bvec = [b_v[pl.ds(k * 16, 16)] for k in range(K)]

    def chunk_body(ci, carry):
        base = base0 + ci * C
        pltpu.sync_copy(tok_ids.at[pl.ds(base, C)], idx_v)
        pltpu.sync_copy(pos_ids.at[pl.ds(base, C)], pidx_v)
        cp1 = pltpu.async_copy(tok_table.at[idx_v], rows_v, sem)
        cp2 = pltpu.async_copy(pos_table.at[pidx_v], prow_v, sem)
        cp1.wait()
        cp2.wait()

        def tok_body(t, carry2):
            x = [rows_v[t, pl.ds(k * 16, 16)] + prow_v[t, pl.ds(k * 16, 16)]
                 for k in range(K)]
            tot = jnp.sum(x[0] + x[1] + x[2] + x[3])
            mean_v = lax.broadcast_in_dim(tot * (1.0 / HID), (16,), ())
            cv = [xx - mean_v for xx in x]
            ss = jnp.sum(cv[0] * cv[0] + cv[1] * cv[1]
                         + cv[2] * cv[2] + cv[3] * cv[3])
            var_v = lax.broadcast_in_dim(ss * (1.0 / HID) + EPS, (16,), ())
            rinv = _rsqrt_nr(var_v)
            for k in range(K):
                out_v[t, pl.ds(k * 16, 16)] = cv[k] * rinv * wvec[k] + bvec[k]
            return carry2

        lax.fori_loop(0, C, tok_body, 0)
        pltpu.sync_copy(out_v, out.at[pl.ds(base, C)])
        return carry

    lax.fori_loop(0, NCHUNK, chunk_body, 0)


@jax.jit
def _run(tok_ids, pos_ids, tok_table, pos_table, ln_w, ln_b):
    mesh = plsc.VectorSubcoreMesh(core_axis_name="c", subcore_axis_name="s")
    f = pl.kernel(
        _body,
        mesh=mesh,
        out_type=jax.ShapeDtypeStruct((N, HID), jnp.float32),
        scratch_types=[
            pltpu.VMEM((C,), jnp.int32),
            pltpu.VMEM((C,), jnp.int32),
            pltpu.VMEM((C, HID), jnp.float32),
            pltpu.VMEM((C, HID), jnp.float32),
            pltpu.VMEM((C, HID), jnp.float32),
            pltpu.VMEM((HID,), jnp.float32),
            pltpu.VMEM((HID,), jnp.float32),
            pltpu.SemaphoreType.DMA,
        ],
    )
    return f(tok_ids, pos_ids, tok_table, pos_table, ln_w, ln_b)


def kernel(token_ids, position_ids, token_table, pos_table, ln_weight, ln_bias):
    tok = token_ids.reshape(-1).astype(jnp.int32)
    pos = position_ids.reshape(-1).astype(jnp.int32)
    out = _run(tok, pos, token_table, pos_table, ln_weight, ln_bias)
    return out.reshape(B, L, HID)


# speedup vs baseline: 1.8725x; 1.8725x over previous
import functools

import jax
import jax.numpy as jnp
from jax import lax
from jax.experimental import pallas as pl
from jax.experimental.pallas import tpu as pltpu
from jax.experimental.pallas import tpu_sc as plsc

B, L, HID = 4096, 200, 64
N = B * L
NC, NS = 2, 16
NW = NC * NS
TOK_PER_W = N // NW
C = 128
NCHUNK = TOK_PER_W // C
EPS = 1e-12
K = HID // 16


def _splat_sum(x, perms):
    dnums = lax.GatherDimensionNumbers(
        offset_dims=(), collapsed_slice_dims=(0,), start_index_map=(0,))
    for p in perms:
        x = x + lax.gather(x, p[:, None], dimension_numbers=dnums,
                           slice_sizes=(1,),
                           mode=lax.GatherScatterMode.PROMISE_IN_BOUNDS)
    return x


def _rsqrt_nr(v):
    i = plsc.bitcast(v, jnp.int32)
    i = jnp.int32(0x5F3759DF) - lax.shift_right_logical(i, jnp.int32(1))
    y = plsc.bitcast(i, jnp.float32)
    for _ in range(3):
        y = y * (1.5 - 0.5 * v * y * y)
    return y


def _body(tok_ids, pos_ids, tok_table, pos_table, ln_w, ln_b, out,
          idx_v, pidx_v, rows_v, prow_v, out_v, w_v, b_v, sem):
    c = lax.axis_index("c")
    s = lax.axis_index("s")
    wid = s * NC + c
    base0 = wid * TOK_PER_W

    pltpu.sync_copy(ln_w, w_v)
    pltpu.sync_copy(ln_b, b_v)
    wvec = [w_v[pl.ds(k * 16, 16)] for k in range(K)]
    bvec = [b_v[pl.ds(k * 16, 16)] for k in range(K)]

    iota = lax.iota(jnp.int32, 16)
    perms = [lax.bitwise_xor(iota, jnp.int32(1 << j)) for j in range(4)]

    def chunk_body(ci, carry):
        base = base0 + ci * C
        pltpu.sync_copy(tok_ids.at[pl.ds(base, C)], idx_v)
        pltpu.sync_copy(pos_ids.at[pl.ds(base, C)], pidx_v)
        cp1 = pltpu.async_copy(tok_table.at[idx_v], rows_v, sem)
        cp2 = pltpu.async_copy(pos_table.at[pidx_v], prow_v, sem)
        cp1.wait()
        cp2.wait()

        def tok_body(t, carry2):
            x = [rows_v[t, pl.ds(k * 16, 16)] + prow_v[t, pl.ds(k * 16, 16)]
                 for k in range(K)]
            tot = _splat_sum(x[0] + x[1] + x[2] + x[3], perms)
            mean_v = tot * (1.0 / HID)
            cv = [xx - mean_v for xx in x]
            ss = _splat_sum(cv[0] * cv[0] + cv[1] * cv[1]
                            + cv[2] * cv[2] + cv[3] * cv[3], perms)
            var_v = ss * (1.0 / HID) + EPS
            rinv = _rsqrt_nr(var_v)
            for k in range(K):
                out_v[t, pl.ds(k * 16, 16)] = cv[k] * rinv * wvec[k] + bvec[k]
            return carry2

        lax.fori_loop(0, C, tok_body, 0)
        pltpu.sync_copy(out_v, out.at[pl.ds(base, C)])
        return carry

    lax.fori_loop(0, NCHUNK, chunk_body, 0)


@jax.jit
def _run(tok_ids, pos_ids, tok_table, pos_table, ln_w, ln_b):
    mesh = plsc.VectorSubcoreMesh(core_axis_name="c", subcore_axis_name="s")
    f = pl.kernel(
        _body,
        mesh=mesh,
        compiler_params=pltpu.CompilerParams(
            needs_layout_passes=False, use_tc_tiling_on_sc=False),
        out_type=jax.ShapeDtypeStruct((N, HID), jnp.float32),
        scratch_types=[
            pltpu.VMEM((C,), jnp.int32),
            pltpu.VMEM((C,), jnp.int32),
            pltpu.VMEM((C, HID), jnp.float32),
            pltpu.VMEM((C, HID), jnp.float32),
            pltpu.VMEM((C, HID), jnp.float32),
            pltpu.VMEM((HID,), jnp.float32),
            pltpu.VMEM((HID,), jnp.float32),
            pltpu.SemaphoreType.DMA,
        ],
    )
    return f(tok_ids, pos_ids, tok_table, pos_table, ln_w, ln_b)


def kernel(token_ids, position_ids, token_table, pos_table, ln_weight, ln_bias):
    tok = token_ids.reshape(-1).astype(jnp.int32)
    pos = position_ids.reshape(-1).astype(jnp.int32)
    out = _run(tok, pos, token_table, pos_table, ln_weight, ln_bias)
    return out.reshape(B, L, HID)


# speedup vs baseline: 2.1050x; 1.1241x over previous
import functools

import jax
import jax.numpy as jnp
from jax import lax
from jax.experimental import pallas as pl
from jax.experimental.pallas import tpu as pltpu
from jax.experimental.pallas import tpu_sc as plsc

B, L, HID = 4096, 200, 64
N = B * L
NC, NS = 2, 16
NW = NC * NS
TOK_PER_W = N // NW
C = 128
NCHUNK = TOK_PER_W // C
NBUF = 2
EPS = 1e-12
K = HID // 16
UNROLL = 8


def _splat_sum(x, perms):
    dnums = lax.GatherDimensionNumbers(
        offset_dims=(), collapsed_slice_dims=(0,), start_index_map=(0,))
    for p in perms:
        x = x + lax.gather(x, p[:, None], dimension_numbers=dnums,
                           slice_sizes=(1,),
                           mode=lax.GatherScatterMode.PROMISE_IN_BOUNDS)
    return x


def _rsqrt_nr(v):
    i = plsc.bitcast(v, jnp.int32)
    i = jnp.int32(0x5F3759DF) - lax.shift_right_logical(i, jnp.int32(1))
    y = plsc.bitcast(i, jnp.float32)
    for _ in range(3):
        y = y * (1.5 - 0.5 * v * y * y)
    return y


def _body(tok_ids, pos_ids, tok_table, pos_table, ln_w, ln_b, out,
          idx_all, pidx_all, rows_v, prow_v, out_v, w_v, b_v, gsems, osems):
    c = lax.axis_index("c")
    s = lax.axis_index("s")
    wid = s * NC + c
    base0 = wid * TOK_PER_W

    pltpu.sync_copy(ln_w, w_v)
    pltpu.sync_copy(ln_b, b_v)
    pltpu.sync_copy(tok_ids.at[wid], idx_all)
    pltpu.sync_copy(pos_ids.at[wid], pidx_all)

    wvec = [w_v[pl.ds(k * 16, 16)] for k in range(K)]
    bvec = [b_v[pl.ds(k * 16, 16)] for k in range(K)]
    iota = lax.iota(jnp.int32, 16)
    perms = [lax.bitwise_xor(iota, jnp.int32(1 << j)) for j in range(4)]

    def issue_gathers(ci, b):
        pltpu.make_async_copy(
            tok_table.at[idx_all.at[ci]], rows_v.at[b], gsems.at[b]).start()
        pltpu.make_async_copy(
            pos_table.at[pidx_all.at[ci]], prow_v.at[b], gsems.at[b]).start()

    def wait_gathers(ci, b):
        pltpu.make_async_copy(
            tok_table.at[idx_all.at[ci]], rows_v.at[b], gsems.at[b]).wait()
        pltpu.make_async_copy(
            pos_table.at[pidx_all.at[ci]], prow_v.at[b], gsems.at[b]).wait()

    for b in range(NBUF):
        issue_gathers(b, b)

    def outer_body(oi, carry):
        for b in range(NBUF):
            ci = oi * NBUF + b
            wait_gathers(ci, b)

            @pl.when(oi > 0)
            def _():
                pltpu.make_async_copy(
                    out_v.at[b], out.at[pl.ds(base0, C)], osems.at[b]).wait()

            rows = rows_v.at[b]
            prow = prow_v.at[b]
            outb = out_v.at[b]

            @plsc.parallel_loop(0, C, unroll=UNROLL)
            def _(t):
                x = [rows[t, pl.ds(k * 16, 16)] + prow[t, pl.ds(k * 16, 16)]
                     for k in range(K)]
                tot = _splat_sum(x[0] + x[1] + x[2] + x[3], perms)
                mean_v = tot * (1.0 / HID)
                cv = [xx - mean_v for xx in x]
                ss = _splat_sum(cv[0] * cv[0] + cv[1] * cv[1]
                                + cv[2] * cv[2] + cv[3] * cv[3], perms)
                var_v = ss * (1.0 / HID) + EPS
                rinv = _rsqrt_nr(var_v)
                for k in range(K):
                    outb[t, pl.ds(k * 16, 16)] = \
                        cv[k] * rinv * wvec[k] + bvec[k]

            pltpu.make_async_copy(
                out_v.at[b], out.at[pl.ds(base0 + ci * C, C)],
                osems.at[b]).start()

            @pl.when(ci + NBUF < NCHUNK)
            def _():
                issue_gathers(ci + NBUF, b)
        return carry

    lax.fori_loop(0, NCHUNK // NBUF, outer_body, 0)

    for b in range(NBUF):
        pltpu.make_async_copy(
            out_v.at[b], out.at[pl.ds(base0, C)], osems.at[b]).wait()


@jax.jit
def _run(tok_ids, pos_ids, tok_table, pos_table, ln_w, ln_b):
    mesh = plsc.VectorSubcoreMesh(core_axis_name="c", subcore_axis_name="s")
    f = pl.kernel(
        _body,
        mesh=mesh,
        compiler_params=pltpu.CompilerParams(
            needs_layout_passes=False, use_tc_tiling_on_sc=False),
        out_type=jax.ShapeDtypeStruct((N, HID), jnp.float32),
        scratch_types=[
            pltpu.VMEM((NCHUNK, C), jnp.int32),
            pltpu.VMEM((NCHUNK, C), jnp.int32),
            pltpu.VMEM((NBUF, C, HID), jnp.float32),
            pltpu.VMEM((NBUF, C, HID), jnp.float32),
            pltpu.VMEM((NBUF, C, HID), jnp.float32),
            pltpu.VMEM((HID,), jnp.float32),
            pltpu.VMEM((HID,), jnp.float32),
            pltpu.SemaphoreType.DMA((NBUF,)),
            pltpu.SemaphoreType.DMA((NBUF,)),
        ],
    )
    return f(tok_ids, pos_ids, tok_table, pos_table, ln_w, ln_b)


def kernel(token_ids, position_ids, token_table, pos_table, ln_weight, ln_bias):
    tok = token_ids.reshape(NW, NCHUNK, C).astype(jnp.int32)
    pos = position_ids.reshape(NW, NCHUNK, C).astype(jnp.int32)
    out = _run(tok, pos, token_table, pos_table, ln_weight, ln_bias)
    return out.reshape(B, L, HID)
